# baseline (speedup 1.0000x reference)
import math
from functools import partial

import jax
import jax.numpy as jnp
from jax.experimental import pallas as pl
from jax.experimental.pallas import tpu as pltpu

_BN_EPS = 1e-5
_SLOPE = 0.1
_VMEM_LIMIT = 100 * 1024 * 1024


def _pick_tile(m, cap):
    for t in (cap, cap // 2, cap // 4, cap // 8, 128, 64, 32, 16, 8):
        if t >= 8 and m % t == 0:
            return t
    return m


def _fused_kernel(xa_ref, xb_ref, w_ref, g_ref, b_ref, o_ref,
                  z_ref, sum_ref, sq_ref, scale_ref, shift_ref,
                  *, tm0, tm1, n_pairs, m):
    i = pl.program_id(0)

    @pl.when(i < n_pairs)
    def _compute():
        @pl.when(i == 0)
        def _init():
            sum_ref[...] = jnp.zeros_like(sum_ref)
            sq_ref[...] = jnp.zeros_like(sq_ref)

        w = w_ref[...]
        za = jnp.dot(xa_ref[...].astype(jnp.bfloat16), w,
                     preferred_element_type=jnp.float32)
        zb = jnp.dot(xb_ref[...].astype(jnp.bfloat16), w,
                     preferred_element_type=jnp.float32)
        z_ref[pl.ds((2 * i) * tm0, tm0), :] = za.astype(jnp.bfloat16)
        z_ref[pl.ds((2 * i + 1) * tm0, tm0), :] = zb.astype(jnp.bfloat16)
        sum_ref[...] += (jnp.sum(za, axis=0, keepdims=True) +
                         jnp.sum(zb, axis=0, keepdims=True))
        sq_ref[...] += (jnp.sum(za * za, axis=0, keepdims=True) +
                        jnp.sum(zb * zb, axis=0, keepdims=True))

    @pl.when(i >= n_pairs)
    def _normalize():
        @pl.when(i == n_pairs)
        def _fold_stats():
            inv_m = 1.0 / m
            mean = sum_ref[...] * inv_m
            var = jnp.maximum(sq_ref[...] * inv_m - mean * mean, 0.0)
            scale_ref[...] = g_ref[...] * jax.lax.rsqrt(var + _BN_EPS)
            shift_ref[...] = b_ref[...] - mean * scale_ref[...]

        j = i - n_pairs
        zt = z_ref[pl.ds(j * tm1, tm1), :].astype(jnp.float32)
        y = zt * scale_ref[...] + shift_ref[...]
        o_ref[...] = jnp.where(y > 0, y, _SLOPE * y)


@jax.jit
def _run(x, weight, gamma, beta):
    B, N, in_dim = x.shape
    out_dim = weight.shape[0]
    M = B * N
    x2 = x.reshape(M, in_dim)
    wt = weight.T.astype(jnp.bfloat16)

    tm0 = _pick_tile(M, 1024)
    if M % (2 * tm0) != 0:
        tm0 = max(tm0 // 2, 8)
    n_pairs = M // (2 * tm0)
    tm1 = 4 * tm0 if M % (4 * tm0) == 0 else tm0
    n1 = M // tm1
    f32 = jnp.float32

    def xa_map(i):
        return (2 * jnp.minimum(i, n_pairs - 1), 0)

    def xb_map(i):
        return (2 * jnp.minimum(i, n_pairs - 1) + 1, 0)

    out = pl.pallas_call(
        partial(_fused_kernel, tm0=tm0, tm1=tm1, n_pairs=n_pairs, m=M),
        out_shape=jax.ShapeDtypeStruct((M, out_dim), x.dtype),
        grid=(n_pairs + n1,),
        in_specs=[pl.BlockSpec((tm0, in_dim), xa_map),
                  pl.BlockSpec((tm0, in_dim), xb_map),
                  pl.BlockSpec((in_dim, out_dim), lambda i: (0, 0)),
                  pl.BlockSpec((1, out_dim), lambda i: (0, 0)),
                  pl.BlockSpec((1, out_dim), lambda i: (0, 0))],
        out_specs=pl.BlockSpec((tm1, out_dim),
                               lambda i: (jnp.maximum(i - n_pairs, 0), 0)),
        scratch_shapes=[pltpu.VMEM((M, out_dim), jnp.bfloat16),
                        pltpu.VMEM((1, out_dim), f32),
                        pltpu.VMEM((1, out_dim), f32),
                        pltpu.VMEM((1, out_dim), f32),
                        pltpu.VMEM((1, out_dim), f32)],
        compiler_params=pltpu.CompilerParams(
            dimension_semantics=("arbitrary",),
            vmem_limit_bytes=_VMEM_LIMIT),
    )(x2, x2, wt, gamma.reshape(1, out_dim).astype(f32),
      beta.reshape(1, out_dim).astype(f32))

    return out.reshape(B, N, out_dim)


def kernel(x, weight, bias, gamma, beta):
    del bias
    return _run(x, weight, gamma, beta)


# speedup vs baseline: 1.0744x; 1.0744x over previous
import math
from functools import partial

import jax
import jax.numpy as jnp
from jax.experimental import pallas as pl
from jax.experimental.pallas import tpu as pltpu

_BN_EPS = 1e-5
_SLOPE = 0.1
_VMEM_LIMIT = 100 * 1024 * 1024
_NSLOTS = 3


def _pick_tile(m):
    for t in (2048, 1024, 512, 256, 128, 64, 32, 16, 8):
        if m % t == 0:
            return t
    return m


def _fused_kernel(xa_ref, xb_ref, w_ref, g_ref, b_ref, o_ref,
                  z_ref, y_ref, sum_ref, sq_ref, scale_ref, shift_ref,
                  out_sem, *, tm0, tm1, n_pairs, n1, m):
    i = pl.program_id(0)

    def out_copy(slot, j):
        return pltpu.make_async_copy(
            y_ref.at[slot], o_ref.at[pl.ds(j * tm1, tm1), :],
            out_sem.at[slot])

    @pl.when(i < n_pairs)
    def _compute():
        @pl.when(i == 0)
        def _init():
            sum_ref[...] = jnp.zeros_like(sum_ref)
            sq_ref[...] = jnp.zeros_like(sq_ref)

        w = w_ref[...]
        za = jnp.dot(xa_ref[...].astype(jnp.bfloat16), w,
                     preferred_element_type=jnp.float32)
        zb = jnp.dot(xb_ref[...].astype(jnp.bfloat16), w,
                     preferred_element_type=jnp.float32)
        z_ref[pl.ds((2 * i) * tm0, tm0), :] = za.astype(jnp.bfloat16)
        z_ref[pl.ds((2 * i + 1) * tm0, tm0), :] = zb.astype(jnp.bfloat16)
        sum_ref[...] += (jnp.sum(za, axis=0, keepdims=True) +
                         jnp.sum(zb, axis=0, keepdims=True))
        sq_ref[...] += (jnp.sum(za * za, axis=0, keepdims=True) +
                        jnp.sum(zb * zb, axis=0, keepdims=True))

    @pl.when(i >= n_pairs)
    def _normalize():
        j = i - n_pairs
        slot = jax.lax.rem(j, _NSLOTS)

        @pl.when(i == n_pairs)
        def _fold_stats():
            inv_m = 1.0 / m
            mean = sum_ref[...] * inv_m
            var = jnp.maximum(sq_ref[...] * inv_m - mean * mean, 0.0)
            scale_ref[...] = g_ref[...] * jax.lax.rsqrt(var + _BN_EPS)
            shift_ref[...] = b_ref[...] - mean * scale_ref[...]

        @pl.when(j >= _NSLOTS)
        def _reclaim():
            out_copy(slot, j - _NSLOTS).wait()

        zt = z_ref[pl.ds(j * tm1, tm1), :].astype(jnp.float32)
        y = zt * scale_ref[...] + shift_ref[...]
        y_ref[slot, :, :] = jnp.where(y > 0, y, _SLOPE * y)
        out_copy(slot, j).start()

        @pl.when(j == n1 - 1)
        def _drain():
            for jj in range(max(0, n1 - _NSLOTS), n1):
                out_copy(jj % _NSLOTS, jj).wait()


@jax.jit
def _run(x, weight, gamma, beta):
    B, N, in_dim = x.shape
    out_dim = weight.shape[0]
    M = B * N
    x2 = x.reshape(M, in_dim)
    wt = weight.T.astype(jnp.bfloat16)

    tm0 = _pick_tile(M)
    if M % (2 * tm0) != 0:
        tm0 = max(tm0 // 2, 8)
    n_pairs = M // (2 * tm0)
    tm1 = tm0
    n1 = M // tm1
    f32 = jnp.float32

    def xa_map(i):
        return (2 * jnp.minimum(i, n_pairs - 1), 0)

    def xb_map(i):
        return (2 * jnp.minimum(i, n_pairs - 1) + 1, 0)

    out = pl.pallas_call(
        partial(_fused_kernel, tm0=tm0, tm1=tm1, n_pairs=n_pairs,
                n1=n1, m=M),
        out_shape=jax.ShapeDtypeStruct((M, out_dim), x.dtype),
        grid=(n_pairs + n1,),
        in_specs=[pl.BlockSpec((tm0, in_dim), xa_map),
                  pl.BlockSpec((tm0, in_dim), xb_map),
                  pl.BlockSpec((in_dim, out_dim), lambda i: (0, 0)),
                  pl.BlockSpec((1, out_dim), lambda i: (0, 0)),
                  pl.BlockSpec((1, out_dim), lambda i: (0, 0))],
        out_specs=pl.BlockSpec(memory_space=pl.ANY),
        scratch_shapes=[pltpu.VMEM((M, out_dim), jnp.bfloat16),
                        pltpu.VMEM((_NSLOTS, tm1, out_dim), f32),
                        pltpu.VMEM((1, out_dim), f32),
                        pltpu.VMEM((1, out_dim), f32),
                        pltpu.VMEM((1, out_dim), f32),
                        pltpu.VMEM((1, out_dim), f32),
                        pltpu.SemaphoreType.DMA((_NSLOTS,))],
        compiler_params=pltpu.CompilerParams(
            dimension_semantics=("arbitrary",),
            vmem_limit_bytes=_VMEM_LIMIT),
    )(x2, x2, wt, gamma.reshape(1, out_dim).astype(f32),
      beta.reshape(1, out_dim).astype(f32))

    return out.reshape(B, N, out_dim)


def kernel(x, weight, bias, gamma, beta):
    del bias
    return _run(x, weight, gamma, beta)


# speedup vs baseline: 1.0912x; 1.0156x over previous
import math
from functools import partial

import jax
import jax.numpy as jnp
from jax.experimental import pallas as pl
from jax.experimental.pallas import tpu as pltpu

_BN_EPS = 1e-5
_SLOPE = 0.1
_VMEM_LIMIT = 100 * 1024 * 1024
_RSLOTS = 5
_WSLOTS = 4


def _pick_tile(m, cap):
    t = cap
    while t >= 8:
        if m % t == 0:
            return t
        t //= 2
    return m


def _fused_kernel(x_hbm, w_ref, g_ref, b_ref, o_hbm,
                  x_buf, y_buf, z_ref, sum_ref, sq_ref, scale_ref, shift_ref,
                  in_sem, out_sem, *, tm_r, n_r, tm_w, n_w, m):

    def in_copy(slot, s):
        return pltpu.make_async_copy(
            x_hbm.at[pl.ds(s * tm_r, tm_r), :], x_buf.at[slot],
            in_sem.at[slot])

    def out_copy(slot, s):
        return pltpu.make_async_copy(
            y_buf.at[slot], o_hbm.at[pl.ds(s * tm_w, tm_w), :],
            out_sem.at[slot])

    sum_ref[...] = jnp.zeros_like(sum_ref)
    sq_ref[...] = jnp.zeros_like(sq_ref)

    for s in range(min(_RSLOTS - 1, n_r)):
        in_copy(s % _RSLOTS, s).start()

    w = w_ref[...]

    def _phase0(s, _):
        slot = jax.lax.rem(s, _RSLOTS)
        in_copy(slot, s).wait()
        z = jnp.dot(x_buf[slot].astype(jnp.bfloat16), w,
                    preferred_element_type=jnp.float32)
        z_ref[pl.ds(s * tm_r, tm_r), :] = z.astype(jnp.bfloat16)
        sum_ref[...] += jnp.sum(z, axis=0, keepdims=True)
        sq_ref[...] += jnp.sum(z * z, axis=0, keepdims=True)

        nxt = s + _RSLOTS - 1
        @pl.when(nxt < n_r)
        def _():
            in_copy(jax.lax.rem(nxt, _RSLOTS), nxt).start()
        return ()

    jax.lax.fori_loop(0, n_r, _phase0, (), unroll=False)

    inv_m = 1.0 / m
    mean = sum_ref[...] * inv_m
    var = jnp.maximum(sq_ref[...] * inv_m - mean * mean, 0.0)
    scale_ref[...] = g_ref[...] * jax.lax.rsqrt(var + _BN_EPS)
    shift_ref[...] = b_ref[...] - mean * scale_ref[...]

    def _phase1(s, _):
        slot = jax.lax.rem(s, _WSLOTS)

        @pl.when(s >= _WSLOTS)
        def _():
            out_copy(slot, s - _WSLOTS).wait()

        zt = z_ref[pl.ds(s * tm_w, tm_w), :].astype(jnp.float32)
        y = zt * scale_ref[...] + shift_ref[...]
        y_buf[slot, :, :] = jnp.where(y > 0, y, _SLOPE * y)
        out_copy(slot, s).start()
        return ()

    jax.lax.fori_loop(0, n_w, _phase1, (), unroll=False)

    for s in range(max(0, n_w - _WSLOTS), n_w):
        out_copy(s % _WSLOTS, s).wait()


@jax.jit
def _run(x, weight, gamma, beta):
    B, N, in_dim = x.shape
    out_dim = weight.shape[0]
    M = B * N
    x2 = x.reshape(M, in_dim)
    wt = weight.T.astype(jnp.bfloat16)

    tm_r = _pick_tile(M, 2048)
    n_r = M // tm_r
    tm_w = _pick_tile(M, 1024)
    n_w = M // tm_w
    f32 = jnp.float32

    out = pl.pallas_call(
        partial(_fused_kernel, tm_r=tm_r, n_r=n_r, tm_w=tm_w, n_w=n_w, m=M),
        out_shape=jax.ShapeDtypeStruct((M, out_dim), x.dtype),
        in_specs=[pl.BlockSpec(memory_space=pl.ANY),
                  pl.BlockSpec((in_dim, out_dim), lambda: (0, 0)),
                  pl.BlockSpec((1, out_dim), lambda: (0, 0)),
                  pl.BlockSpec((1, out_dim), lambda: (0, 0))],
        out_specs=pl.BlockSpec(memory_space=pl.ANY),
        scratch_shapes=[pltpu.VMEM((_RSLOTS, tm_r, in_dim), f32),
                        pltpu.VMEM((_WSLOTS, tm_w, out_dim), f32),
                        pltpu.VMEM((M, out_dim), jnp.bfloat16),
                        pltpu.VMEM((1, out_dim), f32),
                        pltpu.VMEM((1, out_dim), f32),
                        pltpu.VMEM((1, out_dim), f32),
                        pltpu.VMEM((1, out_dim), f32),
                        pltpu.SemaphoreType.DMA((_RSLOTS,)),
                        pltpu.SemaphoreType.DMA((_WSLOTS,))],
        compiler_params=pltpu.CompilerParams(
            vmem_limit_bytes=_VMEM_LIMIT),
    )(x2, wt, gamma.reshape(1, out_dim).astype(f32),
      beta.reshape(1, out_dim).astype(f32))

    return out.reshape(B, N, out_dim)


def kernel(x, weight, bias, gamma, beta):
    del bias
    return _run(x, weight, gamma, beta)
